# initial kernel scaffold (unmeasured)
import functools

import jax
import jax.numpy as jnp
from jax import lax
from jax.experimental import pallas as pl
from jax.experimental.pallas import tpu as pltpu

N_DEV = 4
SEQ = 1024
D = 1024
HQ = 8
DH = 128
BLK = 64
BLOCKS_PER_SHARD = SEQ // BLK
SCALE = 0.08838834764831843
NEG = -1e9


def kernel(x, Wq, K_ext, V_ext, Wo):
    def body(x_ref, wq_ref, k_ref, v_ref, wo_ref, out_ref,
             comm_ref, send_sems, recv_sems, q_scr, acc_scr, m_scr, l_scr):
        my = lax.axis_index("i")
        left = lax.rem(my - 1 + N_DEV, N_DEV)
        right = lax.rem(my + 1, N_DEV)

        barrier_sem = pltpu.get_barrier_semaphore()
        for nbr in [left, right]:
            pl.semaphore_signal(
                barrier_sem, inc=1,
                device_id=(nbr,), device_id_type=pl.DeviceIdType.MESH,
            )
        pl.semaphore_wait(barrier_sem, 2)

        comm_ref[0, :, :D] = jnp.reshape(k_ref[0], (SEQ, D))
        comm_ref[0, :, D:] = jnp.reshape(v_ref[0], (SEQ, D))

        q_scr[...] = lax.dot_general(
            x_ref[0], wq_ref[...], (((1,), (0,)), ((), ())),
            preferred_element_type=jnp.float32,
        )

        m_scr[...] = jnp.full((SEQ, HQ), -1e30, jnp.float32)
        l_scr[...] = jnp.zeros((SEQ, HQ), jnp.float32)
        acc_scr[...] = jnp.zeros((SEQ, D), jnp.float32)

        rows = lax.broadcasted_iota(jnp.int32, (SEQ, SEQ), 0)
        cols = lax.broadcasted_iota(jnp.int32, (SEQ, SEQ), 1)

        def process_chunk(slot, origin):
            qb = my * BLOCKS_PER_SHARD + rows // BLK
            kb = origin * BLOCKS_PER_SHARD + cols // BLK
            mask = (qb == kb) | (kb == 0) | (lax.rem(qb + kb, 3) == 0)
            for h in range(HQ):
                q = q_scr[:, h * DH:(h + 1) * DH]
                k = comm_ref[slot, :, h * DH:(h + 1) * DH]
                v = comm_ref[slot, :, D + h * DH:D + (h + 1) * DH]
                s = lax.dot_general(
                    q, k, (((1,), (1,)), ((), ())),
                    preferred_element_type=jnp.float32,
                ) * SCALE
                s = jnp.where(mask, s, NEG)
                m_prev = m_scr[:, h:h + 1]
                l_prev = l_scr[:, h:h + 1]
                m_new = jnp.maximum(m_prev, jnp.max(s, axis=1, keepdims=True))
                p = jnp.exp(s - m_new)
                corr = jnp.exp(m_prev - m_new)
                l_new = l_prev * corr + jnp.sum(p, axis=1, keepdims=True)
                acc = acc_scr[:, h * DH:(h + 1) * DH]
                acc_new = acc * corr + lax.dot_general(
                    p, v, (((1,), (0,)), ((), ())),
                    preferred_element_type=jnp.float32,
                )
                m_scr[:, h:h + 1] = m_new
                l_scr[:, h:h + 1] = l_new
                acc_scr[:, h * DH:(h + 1) * DH] = acc_new

        for h in range(N_DEV - 1):
            rdma = pltpu.make_async_remote_copy(
                src_ref=comm_ref.at[h],
                dst_ref=comm_ref.at[h + 1],
                send_sem=send_sems.at[h],
                recv_sem=recv_sems.at[h],
                device_id=(right,),
                device_id_type=pl.DeviceIdType.MESH,
            )
            rdma.start()
            process_chunk(h, lax.rem(my - h + N_DEV, N_DEV))
            rdma.wait()
        process_chunk(N_DEV - 1, lax.rem(my - (N_DEV - 1) + N_DEV, N_DEV))

        for h in range(HQ):
            acc_scr[:, h * DH:(h + 1) * DH] = (
                acc_scr[:, h * DH:(h + 1) * DH] / l_scr[:, h:h + 1]
            )
        out_ref[0] = lax.dot_general(
            acc_scr[...], wo_ref[...], (((1,), (0,)), ((), ())),
            preferred_element_type=jnp.float32,
        )

    return pl.pallas_call(
        body,
        out_shape=jax.ShapeDtypeStruct((1, SEQ, D), jnp.float32),
        in_specs=[pl.BlockSpec(memory_space=pltpu.VMEM)] * 5,
        out_specs=pl.BlockSpec(memory_space=pltpu.VMEM),
        scratch_shapes=[
            pltpu.VMEM((N_DEV, SEQ, 2 * D), jnp.float32),
            pltpu.SemaphoreType.DMA((N_DEV - 1,)),
            pltpu.SemaphoreType.DMA((N_DEV - 1,)),
            pltpu.VMEM((SEQ, D), jnp.float32),
            pltpu.VMEM((SEQ, D), jnp.float32),
            pltpu.VMEM((SEQ, HQ), jnp.float32),
            pltpu.VMEM((SEQ, HQ), jnp.float32),
        ],
        compiler_params=pltpu.CompilerParams(collective_id=0),
    )(x, Wq, K_ext, V_ext, Wo)


# baseline (device time: 342371 ns/iter reference)
import jax
import jax.numpy as jnp
from jax import lax
from jax.experimental import pallas as pl
from jax.experimental.pallas import tpu as pltpu

N_DEV = 4
SEQ = 1024
D = 1024
HQ = 8
DH = 128
BLK = 64
BPS = SEQ // BLK
SCALE = 0.08838834764831843
NEG = -1e9


def kernel(x, Wq, K_ext, V_ext, Wo):
    x2 = jnp.reshape(x, (SEQ, D))
    k2 = jnp.reshape(K_ext, (SEQ, D))
    v2 = jnp.reshape(V_ext, (SEQ, D))

    def body(x_ref, wq_ref, k_ref, v_ref, wo_ref, out_ref,
             comm_ref, send_sems, recv_sems, local_sems, credit_sem,
             q_scr, acc_scr, bias_scr, ml_scr):
        my = lax.axis_index("i")
        left = lax.rem(my - 1 + N_DEV, N_DEV)
        right = lax.rem(my + 1, N_DEV)

        cp_k = pltpu.make_async_copy(
            k_ref, comm_ref.at[0, :, 0:D], local_sems.at[0])
        cp_v = pltpu.make_async_copy(
            v_ref, comm_ref.at[0, :, D:2 * D], local_sems.at[1])
        cp_k.start()
        cp_v.start()

        barrier_sem = pltpu.get_barrier_semaphore()
        for nbr in [left, right]:
            pl.semaphore_signal(
                barrier_sem, inc=1,
                device_id=(nbr,), device_id_type=pl.DeviceIdType.MESH,
            )
        pl.semaphore_wait(barrier_sem, 2)

        q_scr[...] = lax.dot_general(
            x_ref[...], wq_ref[...], (((1,), (0,)), ((), ())),
            preferred_element_type=jnp.float32,
        )

        ml_scr[:, 0:HQ] = jnp.full((SEQ, HQ), -1e30, jnp.float32)
        ml_scr[:, HQ:2 * HQ] = jnp.zeros((SEQ, HQ), jnp.float32)
        acc_scr[...] = jnp.zeros((SEQ, D), jnp.float32)

        cp_k.wait()
        cp_v.wait()

        def process_chunk(slot, origin):
            rows = lax.broadcasted_iota(jnp.int32, (SEQ, SEQ), 0)
            cols = lax.broadcasted_iota(jnp.int32, (SEQ, SEQ), 1)
            qb = my * BPS + rows // BLK
            kb = origin * BPS + cols // BLK
            mask = (qb == kb) | (kb == 0) | (lax.rem(qb + kb, 3) == 0)
            bias_scr[...] = jnp.where(mask, 0.0, NEG)
            for h in range(HQ):
                q = q_scr[:, h * DH:(h + 1) * DH]
                k = comm_ref[slot, :, h * DH:(h + 1) * DH]
                v = comm_ref[slot, :, D + h * DH:D + (h + 1) * DH]
                s = lax.dot_general(
                    q, k, (((1,), (1,)), ((), ())),
                    preferred_element_type=jnp.float32,
                ) * SCALE + bias_scr[...]
                m_prev = ml_scr[:, h:h + 1]
                l_prev = ml_scr[:, HQ + h:HQ + h + 1]
                m_new = jnp.maximum(m_prev, jnp.max(s, axis=1, keepdims=True))
                p = jnp.exp(s - m_new)
                corr = jnp.exp(m_prev - m_new)
                l_new = l_prev * corr + jnp.sum(p, axis=1, keepdims=True)
                acc = acc_scr[:, h * DH:(h + 1) * DH]
                acc_new = acc * corr + lax.dot_general(
                    p, v, (((1,), (0,)), ((), ())),
                    preferred_element_type=jnp.float32,
                )
                ml_scr[:, h:h + 1] = m_new
                ml_scr[:, HQ + h:HQ + h + 1] = l_new
                acc_scr[:, h * DH:(h + 1) * DH] = acc_new

        for h in range(N_DEV - 1):
            if h >= 1:
                pl.semaphore_signal(
                    credit_sem, inc=1,
                    device_id=(left,), device_id_type=pl.DeviceIdType.MESH,
                )
                pl.semaphore_wait(credit_sem, 1)
            rdma = pltpu.make_async_remote_copy(
                src_ref=comm_ref.at[h % 2],
                dst_ref=comm_ref.at[(h + 1) % 2],
                send_sem=send_sems.at[h],
                recv_sem=recv_sems.at[h],
                device_id=(right,),
                device_id_type=pl.DeviceIdType.MESH,
            )
            rdma.start()
            process_chunk(h % 2, lax.rem(my - h + N_DEV, N_DEV))
            rdma.wait()
        process_chunk((N_DEV - 1) % 2, lax.rem(my + 1, N_DEV))

        for h in range(HQ):
            acc_scr[:, h * DH:(h + 1) * DH] = (
                acc_scr[:, h * DH:(h + 1) * DH]
                / ml_scr[:, HQ + h:HQ + h + 1]
            )
        out_ref[...] = lax.dot_general(
            acc_scr[...], wo_ref[...], (((1,), (0,)), ((), ())),
            preferred_element_type=jnp.float32,
        )

    out = pl.pallas_call(
        body,
        out_shape=jax.ShapeDtypeStruct((SEQ, D), jnp.float32),
        in_specs=[
            pl.BlockSpec(memory_space=pltpu.VMEM),
            pl.BlockSpec(memory_space=pltpu.VMEM),
            pl.BlockSpec(memory_space=pl.ANY),
            pl.BlockSpec(memory_space=pl.ANY),
            pl.BlockSpec(memory_space=pltpu.VMEM),
        ],
        out_specs=pl.BlockSpec(memory_space=pltpu.VMEM),
        scratch_shapes=[
            pltpu.VMEM((2, SEQ, 2 * D), jnp.float32),
            pltpu.SemaphoreType.DMA((N_DEV - 1,)),
            pltpu.SemaphoreType.DMA((N_DEV - 1,)),
            pltpu.SemaphoreType.DMA((2,)),
            pltpu.SemaphoreType.REGULAR,
            pltpu.VMEM((SEQ, D), jnp.float32),
            pltpu.VMEM((SEQ, D), jnp.float32),
            pltpu.VMEM((SEQ, SEQ), jnp.float32),
            pltpu.VMEM((SEQ, 2 * HQ), jnp.float32),
        ],
        compiler_params=pltpu.CompilerParams(
            collective_id=0,
            vmem_limit_bytes=100 * 1024 * 1024,
        ),
    )(x2, Wq, k2, v2, Wo)
    return jnp.reshape(out, (1, SEQ, D))


# device time: 194766 ns/iter; 1.7579x vs baseline; 1.7579x over previous
import jax
import jax.numpy as jnp
from jax import lax
from jax.experimental import pallas as pl
from jax.experimental.pallas import tpu as pltpu

N_DEV = 4
SEQ = 1024
HALF = SEQ // 2
D = 1024
HQ = 8
DH = 128
BLK = 64
BPS = SEQ // BLK
SCALE = 0.08838834764831843
NEG = -1e9


def kernel(x, Wq, K_ext, V_ext, Wo):
    x2 = jnp.reshape(x, (SEQ, D))
    k2 = jnp.reshape(K_ext, (SEQ, D))
    v2 = jnp.reshape(V_ext, (SEQ, D))

    def body(x_ref, wq_ref, k_ref, v_ref, wo_ref, out_ref,
             comm_ref, sendR, recvR, sendL, recvL, local_sems,
             credit_sems, q_scr, acc_scr, bias_scr, ml_scr):
        my = lax.axis_index("i")
        left = lax.rem(my - 1 + N_DEV, N_DEV)
        right = lax.rem(my + 1, N_DEV)

        cps = [
            pltpu.make_async_copy(
                k_ref.at[0:HALF], comm_ref.at[0, :, 0:D], local_sems.at[0]),
            pltpu.make_async_copy(
                v_ref.at[0:HALF], comm_ref.at[0, :, D:2 * D], local_sems.at[1]),
            pltpu.make_async_copy(
                k_ref.at[HALF:SEQ], comm_ref.at[2, :, 0:D], local_sems.at[2]),
            pltpu.make_async_copy(
                v_ref.at[HALF:SEQ], comm_ref.at[2, :, D:2 * D], local_sems.at[3]),
        ]
        for cp in cps:
            cp.start()

        barrier_sem = pltpu.get_barrier_semaphore()
        for nbr in [left, right]:
            pl.semaphore_signal(
                barrier_sem, inc=1,
                device_id=(nbr,), device_id_type=pl.DeviceIdType.MESH,
            )
        pl.semaphore_wait(barrier_sem, 2)

        q_scr[...] = lax.dot_general(
            x_ref[...], wq_ref[...], (((1,), (0,)), ((), ())),
            preferred_element_type=jnp.float32,
        )

        ml_scr[:, 0:HQ] = jnp.full((SEQ, HQ), -1e30, jnp.float32)
        ml_scr[:, HQ:2 * HQ] = jnp.zeros((SEQ, HQ), jnp.float32)
        acc_scr[...] = jnp.zeros((SEQ, D), jnp.float32)

        for cp in cps:
            cp.wait()

        def fold(slot, origin, half):
            rows = lax.broadcasted_iota(jnp.int32, (SEQ, HALF), 0)
            cols = lax.broadcasted_iota(jnp.int32, (SEQ, HALF), 1)
            qb = my * BPS + rows // BLK
            kb = origin * BPS + (half * HALF // BLK) + cols // BLK
            mask = (qb == kb) | (kb == 0) | (lax.rem(qb + kb, 3) == 0)
            bias_scr[...] = jnp.where(mask, 0.0, NEG)
            for h in range(HQ):
                q = q_scr[:, h * DH:(h + 1) * DH]
                k = comm_ref[slot, :, h * DH:(h + 1) * DH]
                v = comm_ref[slot, :, D + h * DH:D + (h + 1) * DH]
                s = lax.dot_general(
                    q, k, (((1,), (1,)), ((), ())),
                    preferred_element_type=jnp.float32,
                ) * SCALE + bias_scr[...]
                m_prev = ml_scr[:, h:h + 1]
                l_prev = ml_scr[:, HQ + h:HQ + h + 1]
                m_new = jnp.maximum(m_prev, jnp.max(s, axis=1, keepdims=True))
                p = jnp.exp(s - m_new)
                corr = jnp.exp(m_prev - m_new)
                l_new = l_prev * corr + jnp.sum(p, axis=1, keepdims=True)
                acc = acc_scr[:, h * DH:(h + 1) * DH]
                acc_new = acc * corr + lax.dot_general(
                    p, v, (((1,), (0,)), ((), ())),
                    preferred_element_type=jnp.float32,
                )
                ml_scr[:, h:h + 1] = m_new
                ml_scr[:, HQ + h:HQ + h + 1] = l_new
                acc_scr[:, h * DH:(h + 1) * DH] = acc_new

        for h in range(N_DEV - 1):
            if h >= 1:
                pl.semaphore_signal(
                    credit_sems.at[0], inc=1,
                    device_id=(left,), device_id_type=pl.DeviceIdType.MESH,
                )
                pl.semaphore_signal(
                    credit_sems.at[1], inc=1,
                    device_id=(right,), device_id_type=pl.DeviceIdType.MESH,
                )
                pl.semaphore_wait(credit_sems.at[0], 1)
                pl.semaphore_wait(credit_sems.at[1], 1)
            rdma_r = pltpu.make_async_remote_copy(
                src_ref=comm_ref.at[h % 2],
                dst_ref=comm_ref.at[(h + 1) % 2],
                send_sem=sendR.at[h],
                recv_sem=recvR.at[h],
                device_id=(right,),
                device_id_type=pl.DeviceIdType.MESH,
            )
            rdma_l = pltpu.make_async_remote_copy(
                src_ref=comm_ref.at[2 + h % 2],
                dst_ref=comm_ref.at[2 + (h + 1) % 2],
                send_sem=sendL.at[h],
                recv_sem=recvL.at[h],
                device_id=(left,),
                device_id_type=pl.DeviceIdType.MESH,
            )
            rdma_r.start()
            rdma_l.start()
            fold(h % 2, lax.rem(my - h + N_DEV, N_DEV), 0)
            fold(2 + h % 2, lax.rem(my + h, N_DEV), 1)
            rdma_r.wait()
            rdma_l.wait()
        fold((N_DEV - 1) % 2, lax.rem(my + 1, N_DEV), 0)
        fold(2 + (N_DEV - 1) % 2, lax.rem(my - 1 + N_DEV, N_DEV), 1)

        for h in range(HQ):
            acc_scr[:, h * DH:(h + 1) * DH] = (
                acc_scr[:, h * DH:(h + 1) * DH]
                / ml_scr[:, HQ + h:HQ + h + 1]
            )
        out_ref[...] = lax.dot_general(
            acc_scr[...], wo_ref[...], (((1,), (0,)), ((), ())),
            preferred_element_type=jnp.float32,
        )

    out = pl.pallas_call(
        body,
        out_shape=jax.ShapeDtypeStruct((SEQ, D), jnp.float32),
        in_specs=[
            pl.BlockSpec(memory_space=pltpu.VMEM),
            pl.BlockSpec(memory_space=pltpu.VMEM),
            pl.BlockSpec(memory_space=pl.ANY),
            pl.BlockSpec(memory_space=pl.ANY),
            pl.BlockSpec(memory_space=pltpu.VMEM),
        ],
        out_specs=pl.BlockSpec(memory_space=pltpu.VMEM),
        scratch_shapes=[
            pltpu.VMEM((4, HALF, 2 * D), jnp.float32),
            pltpu.SemaphoreType.DMA((N_DEV - 1,)),
            pltpu.SemaphoreType.DMA((N_DEV - 1,)),
            pltpu.SemaphoreType.DMA((N_DEV - 1,)),
            pltpu.SemaphoreType.DMA((N_DEV - 1,)),
            pltpu.SemaphoreType.DMA((4,)),
            pltpu.SemaphoreType.REGULAR((2,)),
            pltpu.VMEM((SEQ, D), jnp.float32),
            pltpu.VMEM((SEQ, D), jnp.float32),
            pltpu.VMEM((SEQ, HALF), jnp.float32),
            pltpu.VMEM((SEQ, 2 * HQ), jnp.float32),
        ],
        compiler_params=pltpu.CompilerParams(
            collective_id=0,
            vmem_limit_bytes=100 * 1024 * 1024,
        ),
    )(x2, Wq, k2, v2, Wo)
    return jnp.reshape(out, (1, SEQ, D))


# device time: 181916 ns/iter; 1.8820x vs baseline; 1.0706x over previous
import jax
import jax.numpy as jnp
from jax import lax
from jax.experimental import pallas as pl
from jax.experimental.pallas import tpu as pltpu

N_DEV = 4
SEQ = 1024
HALF = SEQ // 2
D = 1024
HQ = 8
DH = 128
BLK = 64
BPS = SEQ // BLK
SCALE = 0.08838834764831843
NEG = -1e9


def kernel(x, Wq, K_ext, V_ext, Wo):
    x2 = jnp.reshape(x, (SEQ, D))
    k2 = jnp.reshape(K_ext, (SEQ, D))
    v2 = jnp.reshape(V_ext, (SEQ, D))

    def body(x_ref, wq_ref, k_ref, v_ref, wo_ref, out_ref,
             comm_ref, sendR, recvR, sendL, recvL, local_sems,
             credit_sems, q_scr, acc_scr, bias_scr, ml_scr):
        my = lax.axis_index("i")
        left = lax.rem(my - 1 + N_DEV, N_DEV)
        right = lax.rem(my + 1, N_DEV)

        cps = [
            pltpu.make_async_copy(
                k_ref.at[0:HALF], comm_ref.at[0, :, 0:D], local_sems.at[0]),
            pltpu.make_async_copy(
                v_ref.at[0:HALF], comm_ref.at[0, :, D:2 * D], local_sems.at[1]),
            pltpu.make_async_copy(
                k_ref.at[HALF:SEQ], comm_ref.at[2, :, 0:D], local_sems.at[2]),
            pltpu.make_async_copy(
                v_ref.at[HALF:SEQ], comm_ref.at[2, :, D:2 * D], local_sems.at[3]),
            pltpu.make_async_copy(x_ref, acc_scr, local_sems.at[4]),
        ]
        for cp in cps:
            cp.start()

        barrier_sem = pltpu.get_barrier_semaphore()
        for nbr in [left, right]:
            pl.semaphore_signal(
                barrier_sem, inc=1,
                device_id=(nbr,), device_id_type=pl.DeviceIdType.MESH,
            )
        pl.semaphore_wait(barrier_sem, 2)

        for cp in cps:
            cp.wait()

        def fold(slot, origin, half):
            rows = lax.broadcasted_iota(jnp.int32, (SEQ, HALF), 0)
            cols = lax.broadcasted_iota(jnp.int32, (SEQ, HALF), 1)
            qb = my * BPS + rows // BLK
            kb = origin * BPS + (half * HALF // BLK) + cols // BLK
            mask = (qb == kb) | (kb == 0) | (lax.rem(qb + kb, 3) == 0)
            bias_scr[...] = jnp.where(mask, 0.0, NEG)
            for h in range(HQ):
                q = q_scr[:, h * DH:(h + 1) * DH]
                k = comm_ref[slot, :, h * DH:(h + 1) * DH]
                v = comm_ref[slot, :, D + h * DH:D + (h + 1) * DH]
                s = lax.dot_general(
                    q, k, (((1,), (1,)), ((), ())),
                    preferred_element_type=jnp.float32,
                ) * SCALE + bias_scr[...]
                p = jnp.exp(s)
                l_prev = ml_scr[:, HQ + h:HQ + h + 1]
                ml_scr[:, HQ + h:HQ + h + 1] = (
                    l_prev + jnp.sum(p, axis=1, keepdims=True)
                )
                acc_scr[:, h * DH:(h + 1) * DH] = (
                    acc_scr[:, h * DH:(h + 1) * DH] + lax.dot_general(
                        p, v, (((1,), (0,)), ((), ())),
                        preferred_element_type=jnp.float32,
                    )
                )

        for h in range(N_DEV - 1):
            if h >= 1:
                pl.semaphore_signal(
                    credit_sems.at[0], inc=1,
                    device_id=(left,), device_id_type=pl.DeviceIdType.MESH,
                )
                pl.semaphore_signal(
                    credit_sems.at[1], inc=1,
                    device_id=(right,), device_id_type=pl.DeviceIdType.MESH,
                )
                pl.semaphore_wait(credit_sems.at[0], 1)
                pl.semaphore_wait(credit_sems.at[1], 1)
            rdma_r = pltpu.make_async_remote_copy(
                src_ref=comm_ref.at[h % 2],
                dst_ref=comm_ref.at[(h + 1) % 2],
                send_sem=sendR.at[h],
                recv_sem=recvR.at[h],
                device_id=(right,),
                device_id_type=pl.DeviceIdType.MESH,
            )
            rdma_l = pltpu.make_async_remote_copy(
                src_ref=comm_ref.at[2 + h % 2],
                dst_ref=comm_ref.at[2 + (h + 1) % 2],
                send_sem=sendL.at[h],
                recv_sem=recvL.at[h],
                device_id=(left,),
                device_id_type=pl.DeviceIdType.MESH,
            )
            rdma_r.start()
            rdma_l.start()
            if h == 0:
                q_scr[...] = lax.dot_general(
                    acc_scr[...], wq_ref[...], (((1,), (0,)), ((), ())),
                    preferred_element_type=jnp.float32,
                )
                ml_scr[...] = jnp.zeros((SEQ, 2 * HQ), jnp.float32)
                acc_scr[...] = jnp.zeros((SEQ, D), jnp.float32)
            fold(h % 2, lax.rem(my - h + N_DEV, N_DEV), 0)
            fold(2 + h % 2, lax.rem(my + h, N_DEV), 1)
            rdma_r.wait()
            rdma_l.wait()
        fold((N_DEV - 1) % 2, lax.rem(my + 1, N_DEV), 0)
        fold(2 + (N_DEV - 1) % 2, lax.rem(my - 1 + N_DEV, N_DEV), 1)

        for h in range(HQ):
            acc_scr[:, h * DH:(h + 1) * DH] = (
                acc_scr[:, h * DH:(h + 1) * DH]
                / ml_scr[:, HQ + h:HQ + h + 1]
            )
        q_scr[...] = lax.dot_general(
            acc_scr[...], wo_ref[...], (((1,), (0,)), ((), ())),
            preferred_element_type=jnp.float32,
        )
        cp_out = pltpu.make_async_copy(q_scr, out_ref, local_sems.at[4])
        cp_out.start()
        cp_out.wait()

    out = pl.pallas_call(
        body,
        out_shape=jax.ShapeDtypeStruct((SEQ, D), jnp.float32),
        in_specs=[
            pl.BlockSpec(memory_space=pl.ANY),
            pl.BlockSpec(memory_space=pltpu.VMEM),
            pl.BlockSpec(memory_space=pl.ANY),
            pl.BlockSpec(memory_space=pl.ANY),
            pl.BlockSpec(memory_space=pltpu.VMEM),
        ],
        out_specs=pl.BlockSpec(memory_space=pl.ANY),
        scratch_shapes=[
            pltpu.VMEM((4, HALF, 2 * D), jnp.float32),
            pltpu.SemaphoreType.DMA((N_DEV - 1,)),
            pltpu.SemaphoreType.DMA((N_DEV - 1,)),
            pltpu.SemaphoreType.DMA((N_DEV - 1,)),
            pltpu.SemaphoreType.DMA((N_DEV - 1,)),
            pltpu.SemaphoreType.DMA((5,)),
            pltpu.SemaphoreType.REGULAR((2,)),
            pltpu.VMEM((SEQ, D), jnp.float32),
            pltpu.VMEM((SEQ, D), jnp.float32),
            pltpu.VMEM((SEQ, HALF), jnp.float32),
            pltpu.VMEM((SEQ, 2 * HQ), jnp.float32),
        ],
        compiler_params=pltpu.CompilerParams(
            collective_id=0,
            vmem_limit_bytes=100 * 1024 * 1024,
        ),
    )(x2, Wq, k2, v2, Wo)
    return jnp.reshape(out, (1, SEQ, D))


# device time: 118303 ns/iter; 2.8940x vs baseline; 1.5377x over previous
import jax
import jax.numpy as jnp
from jax import lax
from jax.experimental import pallas as pl
from jax.experimental.pallas import tpu as pltpu

N_DEV = 4
SEQ = 1024
HALF = SEQ // 2
D = 1024
HQ = 8
DH = 128
BLK = 64
BPS = SEQ // BLK
SCALE = 0.08838834764831843
NEG = -1e9


def kernel(x, Wq, K_ext, V_ext, Wo):
    x2 = jnp.reshape(x, (SEQ, D)).astype(jnp.bfloat16)
    k2 = jnp.reshape(K_ext, (SEQ, D)).astype(jnp.bfloat16)
    v2 = jnp.reshape(V_ext, (SEQ, D)).astype(jnp.bfloat16)
    wq2 = Wq.astype(jnp.bfloat16)
    wo2 = Wo.astype(jnp.bfloat16)

    def body(x_ref, wq_ref, k_ref, v_ref, wo_ref, out_ref,
             comm_ref, sendR, recvR, sendL, recvL, local_sems,
             credit_sems, q_scr, acc_scr, bias_scr, l_scr):
        my = lax.axis_index("i")
        left = lax.rem(my - 1 + N_DEV, N_DEV)
        right = lax.rem(my + 1, N_DEV)

        cps = [
            pltpu.make_async_copy(
                k_ref.at[0:HALF], comm_ref.at[0, :, 0:D], local_sems.at[0]),
            pltpu.make_async_copy(
                v_ref.at[0:HALF], comm_ref.at[0, :, D:2 * D], local_sems.at[1]),
            pltpu.make_async_copy(
                k_ref.at[HALF:SEQ], comm_ref.at[2, :, 0:D], local_sems.at[2]),
            pltpu.make_async_copy(
                v_ref.at[HALF:SEQ], comm_ref.at[2, :, D:2 * D], local_sems.at[3]),
        ]
        for cp in cps:
            cp.start()

        barrier_sem = pltpu.get_barrier_semaphore()
        for nbr in [left, right]:
            pl.semaphore_signal(
                barrier_sem, inc=1,
                device_id=(nbr,), device_id_type=pl.DeviceIdType.MESH,
            )
        pl.semaphore_wait(barrier_sem, 2)

        for cp in cps:
            cp.wait()

        def fold(slot, origin, half):
            rows = lax.broadcasted_iota(jnp.int32, (SEQ, HALF), 0)
            cols = lax.broadcasted_iota(jnp.int32, (SEQ, HALF), 1)
            qb = my * BPS + rows // BLK
            kb = origin * BPS + (half * HALF // BLK) + cols // BLK
            mask = (qb == kb) | (kb == 0) | (lax.rem(qb + kb, 3) == 0)
            bias_scr[...] = jnp.where(mask, 0.0, NEG)
            for h in range(HQ):
                q = q_scr[:, h * DH:(h + 1) * DH]
                k = comm_ref[slot, :, h * DH:(h + 1) * DH]
                v = comm_ref[slot, :, D + h * DH:D + (h + 1) * DH]
                s = lax.dot_general(
                    q, k, (((1,), (1,)), ((), ())),
                    preferred_element_type=jnp.float32,
                ) * SCALE + bias_scr[...]
                p_f32 = jnp.exp(s)
                p = p_f32.astype(jnp.bfloat16)
                l_scr[:, h:h + 1] = (
                    l_scr[:, h:h + 1]
                    + jnp.sum(p_f32, axis=1, keepdims=True)
                )
                acc_scr[:, h * DH:(h + 1) * DH] = (
                    acc_scr[:, h * DH:(h + 1) * DH] + lax.dot_general(
                        p, v, (((1,), (0,)), ((), ())),
                        preferred_element_type=jnp.float32,
                    )
                )

        for h in range(N_DEV - 1):
            if h >= 1:
                pl.semaphore_signal(
                    credit_sems.at[0], inc=1,
                    device_id=(left,), device_id_type=pl.DeviceIdType.MESH,
                )
                pl.semaphore_signal(
                    credit_sems.at[1], inc=1,
                    device_id=(right,), device_id_type=pl.DeviceIdType.MESH,
                )
                pl.semaphore_wait(credit_sems.at[0], 1)
                pl.semaphore_wait(credit_sems.at[1], 1)
            rdma_r = pltpu.make_async_remote_copy(
                src_ref=comm_ref.at[h % 2],
                dst_ref=comm_ref.at[(h + 1) % 2],
                send_sem=sendR.at[h],
                recv_sem=recvR.at[h],
                device_id=(right,),
                device_id_type=pl.DeviceIdType.MESH,
            )
            rdma_l = pltpu.make_async_remote_copy(
                src_ref=comm_ref.at[2 + h % 2],
                dst_ref=comm_ref.at[2 + (h + 1) % 2],
                send_sem=sendL.at[h],
                recv_sem=recvL.at[h],
                device_id=(left,),
                device_id_type=pl.DeviceIdType.MESH,
            )
            rdma_r.start()
            rdma_l.start()
            if h == 0:
                q_scr[...] = lax.dot_general(
                    x_ref[...], wq_ref[...], (((1,), (0,)), ((), ())),
                    preferred_element_type=jnp.float32,
                ).astype(jnp.bfloat16)
                l_scr[...] = jnp.zeros((SEQ, HQ), jnp.float32)
                acc_scr[...] = jnp.zeros((SEQ, D), jnp.float32)
            fold(h % 2, lax.rem(my - h + N_DEV, N_DEV), 0)
            fold(2 + h % 2, lax.rem(my + h, N_DEV), 1)
            rdma_r.wait()
            rdma_l.wait()
        fold((N_DEV - 1) % 2, lax.rem(my + 1, N_DEV), 0)
        fold(2 + (N_DEV - 1) % 2, lax.rem(my - 1 + N_DEV, N_DEV), 1)

        for h in range(HQ):
            q_scr[:, h * DH:(h + 1) * DH] = (
                acc_scr[:, h * DH:(h + 1) * DH] / l_scr[:, h:h + 1]
            ).astype(jnp.bfloat16)
        acc_scr[...] = lax.dot_general(
            q_scr[...], wo_ref[...], (((1,), (0,)), ((), ())),
            preferred_element_type=jnp.float32,
        )
        cp_out = pltpu.make_async_copy(acc_scr, out_ref, local_sems.at[0])
        cp_out.start()
        cp_out.wait()

    out = pl.pallas_call(
        body,
        out_shape=jax.ShapeDtypeStruct((SEQ, D), jnp.float32),
        in_specs=[
            pl.BlockSpec(memory_space=pltpu.VMEM),
            pl.BlockSpec(memory_space=pltpu.VMEM),
            pl.BlockSpec(memory_space=pl.ANY),
            pl.BlockSpec(memory_space=pl.ANY),
            pl.BlockSpec(memory_space=pltpu.VMEM),
        ],
        out_specs=pl.BlockSpec(memory_space=pl.ANY),
        scratch_shapes=[
            pltpu.VMEM((4, HALF, 2 * D), jnp.bfloat16),
            pltpu.SemaphoreType.DMA((N_DEV - 1,)),
            pltpu.SemaphoreType.DMA((N_DEV - 1,)),
            pltpu.SemaphoreType.DMA((N_DEV - 1,)),
            pltpu.SemaphoreType.DMA((N_DEV - 1,)),
            pltpu.SemaphoreType.DMA((4,)),
            pltpu.SemaphoreType.REGULAR((2,)),
            pltpu.VMEM((SEQ, D), jnp.bfloat16),
            pltpu.VMEM((SEQ, D), jnp.float32),
            pltpu.VMEM((SEQ, HALF), jnp.float32),
            pltpu.VMEM((SEQ, HQ), jnp.float32),
        ],
        compiler_params=pltpu.CompilerParams(
            collective_id=0,
            vmem_limit_bytes=100 * 1024 * 1024,
        ),
    )(x2, wq2, k2, v2, wo2)
    return jnp.reshape(out, (1, SEQ, D))


# device time: 117893 ns/iter; 2.9041x vs baseline; 1.0035x over previous
import jax
import jax.numpy as jnp
from jax import lax
from jax.experimental import pallas as pl
from jax.experimental.pallas import tpu as pltpu

N_DEV = 4
SEQ = 1024
HALF = SEQ // 2
D = 1024
HQ = 8
DH = 128
BLK = 64
BPS = SEQ // BLK
SCALE = 0.08838834764831843
NEG = -1e9


def kernel(x, Wq, K_ext, V_ext, Wo):
    x2 = jnp.reshape(x, (SEQ, D)).astype(jnp.bfloat16)
    k2 = jnp.reshape(K_ext, (SEQ, D)).astype(jnp.bfloat16)
    v2 = jnp.reshape(V_ext, (SEQ, D)).astype(jnp.bfloat16)
    wq2 = Wq.astype(jnp.bfloat16)
    wo2 = Wo.astype(jnp.bfloat16)

    def body(x_ref, wq_ref, k_ref, v_ref, wo_ref, out_ref,
             comm_ref, sendR, recvR, sendL, recvL, local_sems,
             credit_sems, q_scr, acc_scr, bias_scr, l_scr):
        my = lax.axis_index("i")
        left = lax.rem(my - 1 + N_DEV, N_DEV)
        right = lax.rem(my + 1, N_DEV)

        cps = [
            pltpu.make_async_copy(
                k_ref.at[0:HALF], comm_ref.at[0, :, 0:D], local_sems.at[0]),
            pltpu.make_async_copy(
                v_ref.at[0:HALF], comm_ref.at[0, :, D:2 * D], local_sems.at[1]),
            pltpu.make_async_copy(
                k_ref.at[HALF:SEQ], comm_ref.at[2, :, 0:D], local_sems.at[2]),
            pltpu.make_async_copy(
                v_ref.at[HALF:SEQ], comm_ref.at[2, :, D:2 * D], local_sems.at[3]),
        ]
        for cp in cps:
            cp.start()

        barrier_sem = pltpu.get_barrier_semaphore()
        for nbr in [left, right]:
            pl.semaphore_signal(
                barrier_sem, inc=1,
                device_id=(nbr,), device_id_type=pl.DeviceIdType.MESH,
            )
        pl.semaphore_wait(barrier_sem, 2)

        for cp in cps:
            cp.wait()

        def fold(slot, origin, half):
            rows = lax.broadcasted_iota(jnp.int32, (SEQ, HALF), 0)
            cols = lax.broadcasted_iota(jnp.int32, (SEQ, HALF), 1)
            qb = my * BPS + rows // BLK
            kb = origin * BPS + (half * HALF // BLK) + cols // BLK
            mask = (qb == kb) | (kb == 0) | (lax.rem(qb + kb, 3) == 0)
            bias_scr[...] = jnp.where(mask, 0.0, NEG)
            for h in range(HQ):
                q = q_scr[:, h * DH:(h + 1) * DH]
                k = comm_ref[slot, :, h * DH:(h + 1) * DH]
                v = comm_ref[slot, :, D + h * DH:D + (h + 1) * DH]
                s = lax.dot_general(
                    q, k, (((1,), (1,)), ((), ())),
                    preferred_element_type=jnp.float32,
                ) + bias_scr[...]
                p_f32 = jnp.exp(s)
                p = p_f32.astype(jnp.bfloat16)
                l_scr[:, h:h + 1] = (
                    l_scr[:, h:h + 1]
                    + jnp.sum(p_f32, axis=1, keepdims=True)
                )
                acc_scr[:, h * DH:(h + 1) * DH] = (
                    acc_scr[:, h * DH:(h + 1) * DH] + lax.dot_general(
                        p, v, (((1,), (0,)), ((), ())),
                        preferred_element_type=jnp.float32,
                    )
                )

        for h in range(N_DEV - 1):
            if h >= 1:
                pl.semaphore_signal(
                    credit_sems.at[0], inc=1,
                    device_id=(left,), device_id_type=pl.DeviceIdType.MESH,
                )
                pl.semaphore_signal(
                    credit_sems.at[1], inc=1,
                    device_id=(right,), device_id_type=pl.DeviceIdType.MESH,
                )
                pl.semaphore_wait(credit_sems.at[0], 1)
                pl.semaphore_wait(credit_sems.at[1], 1)
            rdma_r = pltpu.make_async_remote_copy(
                src_ref=comm_ref.at[h % 2],
                dst_ref=comm_ref.at[(h + 1) % 2],
                send_sem=sendR.at[h],
                recv_sem=recvR.at[h],
                device_id=(right,),
                device_id_type=pl.DeviceIdType.MESH,
            )
            rdma_l = pltpu.make_async_remote_copy(
                src_ref=comm_ref.at[2 + h % 2],
                dst_ref=comm_ref.at[2 + (h + 1) % 2],
                send_sem=sendL.at[h],
                recv_sem=recvL.at[h],
                device_id=(left,),
                device_id_type=pl.DeviceIdType.MESH,
            )
            rdma_r.start()
            rdma_l.start()
            if h == 0:
                q_scr[...] = (lax.dot_general(
                    x_ref[...], wq_ref[...], (((1,), (0,)), ((), ())),
                    preferred_element_type=jnp.float32,
                ) * SCALE).astype(jnp.bfloat16)
                l_scr[...] = jnp.zeros((SEQ, HQ), jnp.float32)
                acc_scr[...] = jnp.zeros((SEQ, D), jnp.float32)
            fold(h % 2, lax.rem(my - h + N_DEV, N_DEV), 0)
            fold(2 + h % 2, lax.rem(my + h, N_DEV), 1)
            rdma_r.wait()
            rdma_l.wait()
        fold((N_DEV - 1) % 2, lax.rem(my + 1, N_DEV), 0)
        fold(2 + (N_DEV - 1) % 2, lax.rem(my - 1 + N_DEV, N_DEV), 1)

        for h in range(HQ):
            q_scr[:, h * DH:(h + 1) * DH] = (
                acc_scr[:, h * DH:(h + 1) * DH] / l_scr[:, h:h + 1]
            ).astype(jnp.bfloat16)
        acc_scr[...] = lax.dot_general(
            q_scr[...], wo_ref[...], (((1,), (0,)), ((), ())),
            preferred_element_type=jnp.float32,
        )
        cp_out = pltpu.make_async_copy(acc_scr, out_ref, local_sems.at[0])
        cp_out.start()
        cp_out.wait()

    out = pl.pallas_call(
        body,
        out_shape=jax.ShapeDtypeStruct((SEQ, D), jnp.float32),
        in_specs=[
            pl.BlockSpec(memory_space=pltpu.VMEM),
            pl.BlockSpec(memory_space=pltpu.VMEM),
            pl.BlockSpec(memory_space=pl.ANY),
            pl.BlockSpec(memory_space=pl.ANY),
            pl.BlockSpec(memory_space=pltpu.VMEM),
        ],
        out_specs=pl.BlockSpec(memory_space=pl.ANY),
        scratch_shapes=[
            pltpu.VMEM((4, HALF, 2 * D), jnp.bfloat16),
            pltpu.SemaphoreType.DMA((N_DEV - 1,)),
            pltpu.SemaphoreType.DMA((N_DEV - 1,)),
            pltpu.SemaphoreType.DMA((N_DEV - 1,)),
            pltpu.SemaphoreType.DMA((N_DEV - 1,)),
            pltpu.SemaphoreType.DMA((4,)),
            pltpu.SemaphoreType.REGULAR((2,)),
            pltpu.VMEM((SEQ, D), jnp.bfloat16),
            pltpu.VMEM((SEQ, D), jnp.float32),
            pltpu.VMEM((SEQ, HALF), jnp.float32),
            pltpu.VMEM((SEQ, HQ), jnp.float32),
        ],
        compiler_params=pltpu.CompilerParams(
            collective_id=0,
            vmem_limit_bytes=100 * 1024 * 1024,
        ),
    )(x2, wq2, k2, v2, wo2)
    return jnp.reshape(out, (1, SEQ, D))


# device time: 112636 ns/iter; 3.0396x vs baseline; 1.0467x over previous
import jax
import jax.numpy as jnp
from jax import lax
from jax.experimental import pallas as pl
from jax.experimental.pallas import tpu as pltpu

N_DEV = 4
SEQ = 1024
HALF = SEQ // 2
D = 1024
HQ = 8
DH = 128
BLK = 64
BPS = SEQ // BLK
SCALE = 0.08838834764831843
NEG = -1e9


def kernel(x, Wq, K_ext, V_ext, Wo):
    x2 = jnp.reshape(x, (SEQ, D)).astype(jnp.bfloat16)
    k2 = jnp.reshape(K_ext, (SEQ, D)).astype(jnp.bfloat16)
    v2 = jnp.reshape(V_ext, (SEQ, D)).astype(jnp.bfloat16)
    wq2 = Wq.astype(jnp.bfloat16)
    wo2 = Wo.astype(jnp.bfloat16)

    def body(x_ref, wq_ref, k_ref, v_ref, wo_ref, out_ref,
             comm_ref, sendR, recvR, sendL, recvL, local_sems,
             credit_sems, q_scr, acc_scr, bias_scr, l_scr):
        my = lax.axis_index("i")
        left = lax.rem(my - 1 + N_DEV, N_DEV)
        right = lax.rem(my + 1, N_DEV)

        cps = [
            pltpu.make_async_copy(
                k_ref.at[0:HALF], comm_ref.at[0, :, 0:D], local_sems.at[0]),
            pltpu.make_async_copy(
                v_ref.at[0:HALF], comm_ref.at[0, :, D:2 * D], local_sems.at[1]),
            pltpu.make_async_copy(
                k_ref.at[HALF:SEQ], comm_ref.at[2, :, 0:D], local_sems.at[2]),
            pltpu.make_async_copy(
                v_ref.at[HALF:SEQ], comm_ref.at[2, :, D:2 * D], local_sems.at[3]),
        ]
        for cp in cps:
            cp.start()

        barrier_sem = pltpu.get_barrier_semaphore()
        for nbr in [left, right]:
            pl.semaphore_signal(
                barrier_sem, inc=1,
                device_id=(nbr,), device_id_type=pl.DeviceIdType.MESH,
            )
        pl.semaphore_wait(barrier_sem, 2)

        for cp in cps:
            cp.wait()

        def fold(slot, origin, half, r0=0, nr=HALF):
            rows = lax.broadcasted_iota(jnp.int32, (SEQ, nr), 0)
            cols = lax.broadcasted_iota(jnp.int32, (SEQ, nr), 1)
            qb = my * BPS + rows // BLK
            kb = (origin * BPS + (half * HALF + r0) // BLK) + cols // BLK
            mask = (qb == kb) | (kb == 0) | (lax.rem(qb + kb, 3) == 0)
            bias_scr[:, 0:nr] = jnp.where(mask, 0.0, NEG)
            for h in range(HQ):
                q = q_scr[:, h * DH:(h + 1) * DH]
                k = comm_ref[slot, r0:r0 + nr, h * DH:(h + 1) * DH]
                v = comm_ref[slot, r0:r0 + nr, D + h * DH:D + (h + 1) * DH]
                s = lax.dot_general(
                    q, k, (((1,), (1,)), ((), ())),
                    preferred_element_type=jnp.float32,
                ) + bias_scr[:, 0:nr]
                p_f32 = jnp.exp(s)
                p = p_f32.astype(jnp.bfloat16)
                l_scr[:, h:h + 1] = (
                    l_scr[:, h:h + 1]
                    + jnp.sum(p_f32, axis=1, keepdims=True)
                )
                acc_scr[:, h * DH:(h + 1) * DH] = (
                    acc_scr[:, h * DH:(h + 1) * DH] + lax.dot_general(
                        p, v, (((1,), (0,)), ((), ())),
                        preferred_element_type=jnp.float32,
                    )
                )

        QTR = HALF // 2
        for h in range(N_DEV - 1):
            if h >= 1:
                pl.semaphore_signal(
                    credit_sems.at[0], inc=1,
                    device_id=(left,), device_id_type=pl.DeviceIdType.MESH,
                )
                pl.semaphore_signal(
                    credit_sems.at[1], inc=1,
                    device_id=(right,), device_id_type=pl.DeviceIdType.MESH,
                )
                pl.semaphore_wait(credit_sems.at[0], 1)
                pl.semaphore_wait(credit_sems.at[1], 1)
            if h < 2:
                rdmas = [
                    pltpu.make_async_remote_copy(
                        src_ref=comm_ref.at[h % 2],
                        dst_ref=comm_ref.at[(h + 1) % 2],
                        send_sem=sendR.at[h],
                        recv_sem=recvR.at[h],
                        device_id=(right,),
                        device_id_type=pl.DeviceIdType.MESH,
                    ),
                    pltpu.make_async_remote_copy(
                        src_ref=comm_ref.at[2 + h % 2],
                        dst_ref=comm_ref.at[2 + (h + 1) % 2],
                        send_sem=sendL.at[h],
                        recv_sem=recvL.at[h],
                        device_id=(left,),
                        device_id_type=pl.DeviceIdType.MESH,
                    ),
                ]
            else:
                rdmas = []
                for sub in range(2):
                    r0 = sub * QTR
                    rdmas.append(pltpu.make_async_remote_copy(
                        src_ref=comm_ref.at[h % 2, r0:r0 + QTR],
                        dst_ref=comm_ref.at[(h + 1) % 2, r0:r0 + QTR],
                        send_sem=sendR.at[h + sub],
                        recv_sem=recvR.at[h + sub],
                        device_id=(right,),
                        device_id_type=pl.DeviceIdType.MESH,
                    ))
                    rdmas.append(pltpu.make_async_remote_copy(
                        src_ref=comm_ref.at[2 + h % 2, r0:r0 + QTR],
                        dst_ref=comm_ref.at[2 + (h + 1) % 2, r0:r0 + QTR],
                        send_sem=sendL.at[h + sub],
                        recv_sem=recvL.at[h + sub],
                        device_id=(left,),
                        device_id_type=pl.DeviceIdType.MESH,
                    ))
            for r in rdmas:
                r.start()
            if h == 0:
                q_scr[...] = (lax.dot_general(
                    x_ref[...], wq_ref[...], (((1,), (0,)), ((), ())),
                    preferred_element_type=jnp.float32,
                ) * SCALE).astype(jnp.bfloat16)
                l_scr[...] = jnp.zeros((SEQ, HQ), jnp.float32)
                acc_scr[...] = jnp.zeros((SEQ, D), jnp.float32)
            fold(h % 2, lax.rem(my - h + N_DEV, N_DEV), 0)
            fold(2 + h % 2, lax.rem(my + h, N_DEV), 1)
            if h < 2:
                for r in rdmas:
                    r.wait()
        rdmas[0].wait()
        rdmas[1].wait()
        fold((N_DEV - 1) % 2, lax.rem(my + 1, N_DEV), 0, 0, QTR)
        fold(2 + (N_DEV - 1) % 2, lax.rem(my - 1 + N_DEV, N_DEV), 1, 0, QTR)
        rdmas[2].wait()
        rdmas[3].wait()
        fold((N_DEV - 1) % 2, lax.rem(my + 1, N_DEV), 0, QTR, QTR)
        fold(2 + (N_DEV - 1) % 2, lax.rem(my - 1 + N_DEV, N_DEV), 1, QTR, QTR)

        for h in range(HQ):
            q_scr[:, h * DH:(h + 1) * DH] = (
                acc_scr[:, h * DH:(h + 1) * DH] / l_scr[:, h:h + 1]
            ).astype(jnp.bfloat16)
        acc_scr[...] = lax.dot_general(
            q_scr[...], wo_ref[...], (((1,), (0,)), ((), ())),
            preferred_element_type=jnp.float32,
        )
        cp_out = pltpu.make_async_copy(acc_scr, out_ref, local_sems.at[0])
        cp_out.start()
        cp_out.wait()

    out = pl.pallas_call(
        body,
        out_shape=jax.ShapeDtypeStruct((SEQ, D), jnp.float32),
        in_specs=[
            pl.BlockSpec(memory_space=pltpu.VMEM),
            pl.BlockSpec(memory_space=pltpu.VMEM),
            pl.BlockSpec(memory_space=pl.ANY),
            pl.BlockSpec(memory_space=pl.ANY),
            pl.BlockSpec(memory_space=pltpu.VMEM),
        ],
        out_specs=pl.BlockSpec(memory_space=pl.ANY),
        scratch_shapes=[
            pltpu.VMEM((4, HALF, 2 * D), jnp.bfloat16),
            pltpu.SemaphoreType.DMA((N_DEV,)),
            pltpu.SemaphoreType.DMA((N_DEV,)),
            pltpu.SemaphoreType.DMA((N_DEV,)),
            pltpu.SemaphoreType.DMA((N_DEV,)),
            pltpu.SemaphoreType.DMA((4,)),
            pltpu.SemaphoreType.REGULAR((2,)),
            pltpu.VMEM((SEQ, D), jnp.bfloat16),
            pltpu.VMEM((SEQ, D), jnp.float32),
            pltpu.VMEM((SEQ, HALF), jnp.float32),
            pltpu.VMEM((SEQ, HQ), jnp.float32),
        ],
        compiler_params=pltpu.CompilerParams(
            collective_id=0,
            vmem_limit_bytes=100 * 1024 * 1024,
        ),
    )(x2, wq2, k2, v2, wo2)
    return jnp.reshape(out, (1, SEQ, D))
